# baseline (speedup 1.0000x reference)
import functools

import jax
import jax.numpy as jnp
import numpy as np
from jax import lax
from jax.experimental import pallas as pl
from jax.experimental.pallas import tpu as pltpu
from jax.experimental.pallas import tpu_sc as plsc

_LANES = 16
_C = 128
_NBUF = 5


def _threefry2x32_np(k1, k2, x1, x2):
    rot = [(13, 15, 26, 6), (17, 29, 16, 24)]
    ks = [np.uint32(k1), np.uint32(k2),
          np.uint32(np.uint32(k1) ^ np.uint32(k2) ^ np.uint32(0x1BD11BDA))]
    x0 = (x1 + ks[0]).astype(np.uint32)
    x1v = (x2 + ks[1]).astype(np.uint32)
    kseq = [ks[1], ks[2], ks[0]]
    rots = [rot[0], rot[1]]
    for i in range(5):
        for r in rots[0]:
            x0 = (x0 + x1v).astype(np.uint32)
            x1v = ((x1v << np.uint32(r)) | (x1v >> np.uint32(32 - r))).astype(np.uint32)
            x1v = x0 ^ x1v
        x0 = (x0 + kseq[0]).astype(np.uint32)
        x1v = (x1v + kseq[1] + np.uint32(i + 1)).astype(np.uint32)
        kseq = kseq[1:] + kseq[:1]
        rots = rots[1:] + rots[:1]
    return x0, x1v


def _random_bits_np(k, n, partitionable):
    if partitionable:
        b1, b2 = _threefry2x32_np(
            k[0], k[1], np.zeros(n, np.uint32), np.arange(n, dtype=np.uint32))
        return b1 ^ b2
    cnt = np.arange(n, dtype=np.uint32)
    if n % 2:
        cnt = np.append(cnt, np.uint32(0))
    h = len(cnt) // 2
    b1, b2 = _threefry2x32_np(k[0], k[1], cnt[:h], cnt[h:])
    return np.concatenate([b1, b2])[:n]


def _split_np(k1, k2, partitionable):
    if partitionable:
        b1, b2 = _threefry2x32_np(
            k1, k2, np.zeros(2, np.uint32), np.arange(2, dtype=np.uint32))
        return [(b1[0], b2[0]), (b1[1], b2[1])]
    b1, b2 = _threefry2x32_np(
        k1, k2, np.array([0, 1], np.uint32), np.array([2, 3], np.uint32))
    o = np.concatenate([b1, b2])
    return [(o[0], o[1]), (o[2], o[3])]


@functools.lru_cache(maxsize=None)
def _comp_ind_np(batch_size: int, num_samples: int, num_comp: int):
    partitionable = bool(jax.config.jax_threefry_partitionable)
    n = batch_size * num_samples
    khi, klo = _split_np(np.uint32(0), np.uint32(42), partitionable)
    hi_bits = _random_bits_np(khi, n, partitionable)
    lo_bits = _random_bits_np(klo, n, partitionable)
    span = np.uint32(num_comp)
    mult = np.uint32(((2 ** 16 % num_comp) ** 2 % 2 ** 32) % num_comp)
    off = ((hi_bits % span) * mult + (lo_bits % span)).astype(np.uint32) % span
    return off.astype(np.int32)


@functools.lru_cache(maxsize=None)
def _build_sc_kernel(n_rows: int, d: int, num_comp: int):
    info = plsc.get_sparse_core_info()
    nc, ns = info.num_cores, info.num_subcores
    nw = nc * ns
    assert n_rows % (nw * _C * _NBUF) == 0, (n_rows, nw)
    rows_pw = n_rows // nw
    n_chunks = rows_pw // _C
    d_vecs = d // _LANES

    mesh = plsc.VectorSubcoreMesh(core_axis_name="c", subcore_axis_name="s")

    @functools.partial(
        pl.kernel,
        out_type=jax.ShapeDtypeStruct((n_rows, d), jnp.float32),
        mesh=mesh,
        compiler_params=pltpu.CompilerParams(use_tc_tiling_on_sc=False),
        scratch_types=[
            pltpu.VMEM((_NBUF, _C), jnp.int32),
            pltpu.VMEM((_NBUF, _C, d), jnp.float32),
            pltpu.VMEM((_NBUF, _C, d), jnp.float32),
            pltpu.SemaphoreType.DMA((_NBUF,)),
            pltpu.SemaphoreType.DMA((_NBUF,)),
            pltpu.SemaphoreType.DMA((_NBUF,)),
            pltpu.SemaphoreType.DMA((_NBUF,)),
        ],
    )
    def gmm(idx_hbm, noise_hbm, means_hbm, out_hbm,
            idx_v, noise_v, gath_v, idx_sem, gat_sem, noi_sem, out_sem):
        wid = lax.axis_index("s") * nc + lax.axis_index("c")
        base = wid * rows_pw

        def idx_copy(k, b):
            return pltpu.make_async_copy(
                idx_hbm.at[pl.ds(base + k * _C, _C)], idx_v.at[b], idx_sem.at[b])

        def gat_copy(b):
            return pltpu.make_async_copy(
                means_hbm.at[idx_v.at[b]], gath_v.at[b], gat_sem.at[b])

        def noi_copy(k, b):
            return pltpu.make_async_copy(
                noise_hbm.at[pl.ds(base + k * _C, _C)], noise_v.at[b], noi_sem.at[b])

        def out_copy(k, b):
            return pltpu.make_async_copy(
                gath_v.at[b], out_hbm.at[pl.ds(base + k * _C, _C)], out_sem.at[b])

        idx_copy(0, 0).start()
        idx_copy(1, 1).start()
        idx_copy(0, 0).wait()
        gat_copy(0).start()
        noi_copy(0, 0).start()

        def outer(g, carry):
            k0 = g * _NBUF
            for b in range(_NBUF):
                k = k0 + b
                b1 = (b + 1) % _NBUF
                b2 = (b + 2) % _NBUF

                @pl.when(k + 2 < n_chunks)
                def _():
                    idx_copy(k + 2, b2).start()

                @pl.when(k + 1 < n_chunks)
                def _():
                    @pl.when(k + 1 >= _NBUF)
                    def _():
                        out_copy(k + 1 - _NBUF, b1).wait()
                    idx_copy(k + 1, b1).wait()
                    gat_copy(b1).start()
                    noi_copy(k + 1, b1).start()

                gat_copy(b).wait()
                noi_copy(k, b).wait()

                def row(i, c2):
                    for j in range(d_vecs):
                        sl = pl.ds(j * _LANES, _LANES)
                        gath_v[b, i, sl] = gath_v[b, i, sl] + noise_v[b, i, sl] * 0.1
                    return c2

                lax.fori_loop(0, _C, row, carry, unroll=4)
                out_copy(k, b).start()
            return carry

        lax.fori_loop(0, n_chunks // _NBUF, outer, 0)

        for b in range(_NBUF):
            out_copy(n_chunks - _NBUF + b, b).wait()

    return gmm


def kernel(input, noise, target_size, means):
    del input, target_size
    b, s, d = noise.shape
    n = b * s
    idx = jnp.asarray(_comp_ind_np(b, s, means.shape[0]))
    out = _build_sc_kernel(n, d, means.shape[0])(
        idx, noise.reshape(n, d), means
    )
    return out.reshape(b, s, d)


# speedup vs baseline: 1.2338x; 1.2338x over previous
import functools

import jax
import jax.numpy as jnp
import numpy as np
from jax import lax
from jax.experimental import pallas as pl
from jax.experimental.pallas import tpu as pltpu
from jax.experimental.pallas import tpu_sc as plsc

_LANES = 16
_NB = 4
_NBUF = 2


def _threefry2x32_np(k1, k2, x1, x2):
    rot = [(13, 15, 26, 6), (17, 29, 16, 24)]
    ks = [np.uint32(k1), np.uint32(k2),
          np.uint32(np.uint32(k1) ^ np.uint32(k2) ^ np.uint32(0x1BD11BDA))]
    x0 = (x1 + ks[0]).astype(np.uint32)
    x1v = (x2 + ks[1]).astype(np.uint32)
    kseq = [ks[1], ks[2], ks[0]]
    rots = [rot[0], rot[1]]
    for i in range(5):
        for r in rots[0]:
            x0 = (x0 + x1v).astype(np.uint32)
            x1v = ((x1v << np.uint32(r)) | (x1v >> np.uint32(32 - r))).astype(np.uint32)
            x1v = x0 ^ x1v
        x0 = (x0 + kseq[0]).astype(np.uint32)
        x1v = (x1v + kseq[1] + np.uint32(i + 1)).astype(np.uint32)
        kseq = kseq[1:] + kseq[:1]
        rots = rots[1:] + rots[:1]
    return x0, x1v


def _random_bits_np(k, n, partitionable):
    if partitionable:
        b1, b2 = _threefry2x32_np(
            k[0], k[1], np.zeros(n, np.uint32), np.arange(n, dtype=np.uint32))
        return b1 ^ b2
    cnt = np.arange(n, dtype=np.uint32)
    if n % 2:
        cnt = np.append(cnt, np.uint32(0))
    h = len(cnt) // 2
    b1, b2 = _threefry2x32_np(k[0], k[1], cnt[:h], cnt[h:])
    return np.concatenate([b1, b2])[:n]


def _split_np(k1, k2, partitionable):
    if partitionable:
        b1, b2 = _threefry2x32_np(
            k1, k2, np.zeros(2, np.uint32), np.arange(2, dtype=np.uint32))
        return [(b1[0], b2[0]), (b1[1], b2[1])]
    b1, b2 = _threefry2x32_np(
        k1, k2, np.array([0, 1], np.uint32), np.array([2, 3], np.uint32))
    o = np.concatenate([b1, b2])
    return [(o[0], o[1]), (o[2], o[3])]


@functools.lru_cache(maxsize=None)
def _comp_ind_np(batch_size: int, num_samples: int, num_comp: int):
    partitionable = bool(jax.config.jax_threefry_partitionable)
    n = batch_size * num_samples
    khi, klo = _split_np(np.uint32(0), np.uint32(42), partitionable)
    hi_bits = _random_bits_np(khi, n, partitionable)
    lo_bits = _random_bits_np(klo, n, partitionable)
    span = np.uint32(num_comp)
    mult = np.uint32(((2 ** 16 % num_comp) ** 2 % 2 ** 32) % num_comp)
    off = ((hi_bits % span) * mult + (lo_bits % span)).astype(np.uint32) % span
    return off.astype(np.int32)


@functools.lru_cache(maxsize=None)
def _build_sc_kernel(batch: int, nsamp: int, d: int, num_comp: int, dpad: int):
    info = plsc.get_sparse_core_info()
    nc, ns = info.num_cores, info.num_subcores
    nw = nc * ns
    assert batch % (nw * _NB * _NBUF) == 0, (batch, nw)
    b_pw = batch // nw
    n_chunks = b_pw // _NB
    rows_pc = _NB * nsamp
    d_vecs = d // _LANES
    subs = []
    off = 0
    while off < rows_pc:
        sz = min(128, rows_pc - off)
        subs.append((off, sz))
        off += sz
    assert all(o % 8 == 0 and s % 8 == 0 for o, s in subs), subs

    mesh = plsc.VectorSubcoreMesh(core_axis_name="c", subcore_axis_name="s")

    @functools.partial(
        pl.kernel,
        out_type=jax.ShapeDtypeStruct((batch, nsamp, d), jnp.float32),
        mesh=mesh,
        compiler_params=pltpu.CompilerParams(use_tc_tiling_on_sc=True),
        scratch_types=[
            pltpu.VMEM((_NBUF, len(subs), 128), jnp.int32),
            pltpu.VMEM((_NBUF, _NB, nsamp, d), jnp.float32),
            pltpu.VMEM((_NBUF, rows_pc, dpad), jnp.float32),
            pltpu.SemaphoreType.DMA((_NBUF,)),
            pltpu.SemaphoreType.DMA((_NBUF,)),
            pltpu.SemaphoreType.DMA((_NBUF,)),
            pltpu.SemaphoreType.DMA((_NBUF,)),
        ],
    )
    def gmm(idx_hbm, noise_hbm, means_hbm, out_hbm,
            idx_v, noise_v, gath_v, idx_sem, gat_sem, noi_sem, out_sem):
        wid = lax.axis_index("s") * nc + lax.axis_index("c")
        base_b = wid * b_pw

        def idx_copies(k, s):
            r0 = (base_b + k * _NB) * nsamp
            return [
                pltpu.make_async_copy(
                    idx_hbm.at[pl.ds(r0 + o, sz)],
                    idx_v.at[s, j, pl.ds(0, sz)],
                    idx_sem.at[s])
                for j, (o, sz) in enumerate(subs)
            ]

        def gat_copies(s):
            return [
                pltpu.make_async_copy(
                    means_hbm.at[idx_v.at[s, j, pl.ds(0, sz)]],
                    gath_v.at[s, pl.ds(o, sz)],
                    gat_sem.at[s])
                for j, (o, sz) in enumerate(subs)
            ]

        def noi_copy(k, s):
            return pltpu.make_async_copy(
                noise_hbm.at[pl.ds(base_b + k * _NB, _NB)],
                noise_v.at[s], noi_sem.at[s])

        def out_copy(k, s):
            return pltpu.make_async_copy(
                noise_v.at[s], out_hbm.at[pl.ds(base_b + k * _NB, _NB)],
                out_sem.at[s])

        def start(cs):
            for c in cs:
                c.start()

        def wait(cs):
            for c in cs:
                c.wait()

        start(idx_copies(0, 0))
        wait(idx_copies(0, 0))
        start(gat_copies(0))
        noi_copy(0, 0).start()
        start(idx_copies(1, 1))

        def outer(g, carry):
            k0 = g * _NBUF
            for sl in range(_NBUF):
                k = k0 + sl
                s1 = (sl + 1) % _NBUF

                wait(gat_copies(sl))
                noi_copy(k, sl).wait()

                @pl.when(k + 1 < n_chunks)
                def _():
                    @pl.when(k >= 1)
                    def _():
                        out_copy(k - 1, s1).wait()
                    wait(idx_copies(k + 1, s1))
                    start(gat_copies(s1))
                    noi_copy(k + 1, s1).start()

                    @pl.when(k + 2 < n_chunks)
                    def _():
                        start(idx_copies(k + 2, sl))

                def fma(i, c2):
                    for bb in range(_NB):
                        for j in range(d_vecs):
                            v = pl.ds(j * _LANES, _LANES)
                            noise_v[sl, bb, i, v] = (
                                noise_v[sl, bb, i, v] * 0.1
                                + gath_v[sl, bb * nsamp + i, v])
                    return c2

                lax.fori_loop(0, nsamp, fma, carry, unroll=2)
                out_copy(k, sl).start()
            return carry

        lax.fori_loop(0, n_chunks // _NBUF, outer, 0)

        for sl in range(_NBUF):
            out_copy(n_chunks - _NBUF + sl, sl).wait()

    return gmm


def kernel(input, noise, target_size, means):
    del input, target_size
    b, s, d = noise.shape
    num_comp = means.shape[0]
    idx = jnp.asarray(_comp_ind_np(b, s, num_comp))
    dpad = 128
    means_p = jnp.pad(means, ((0, 0), (0, dpad - d)))
    return _build_sc_kernel(b, s, d, num_comp, dpad)(idx, noise, means_p)
